# Initial kernel scaffold; baseline (speedup 1.0000x reference)
import jax
import jax.numpy as jnp
from jax.experimental import pallas as pl


def kernel(x, edge_index, edge_attr, em_res_w, em_res_b, em_w1, em_b1, em_w2, em_b2, em_ln_g, em_ln_b, n1_w1, n1_b1, n1_w2, n1_b2, n1_ln_g, n1_ln_b, n2_res_w, n2_res_b, n2_w1, n2_b1, n2_w2, n2_b2, n2_ln_g, n2_ln_b):
    raise NotImplementedError("write your pallas kernel here")


# speedup vs baseline: 2.1045x; 2.1045x over previous
import functools

import jax
import jax.numpy as jnp
from jax import lax
from jax.experimental import pallas as pl
from jax.experimental.pallas import tpu as pltpu
from jax.experimental.pallas import tpu_sc as plsc

L = 4
C = 128
EC = 2
H = 128
N_NODES = 48008
N_INPUT = 41162
N_EDGES = 432072

N_PAD = 48128
E_PAD = 442368
DUMP = N_NODES
EB = 1024
NB = 512
NW = 32
EPW = E_PAD // NW
CG = 512
CS = 512
EPT = E_PAD // 16
RPT = N_PAD // 16
_ZCH = ((0, 1024), (1024, 1024), (2048, 960))

_f32 = jnp.float32
_SC_PARAMS = pltpu.CompilerParams(use_tc_tiling_on_sc=False)


def _gather_body(nodes_hbm, row_hbm, col_hbm, src_out, dst_out, idx_v, rows_v,
                 sem):
    w = lax.axis_index("s") * 2 + lax.axis_index("c")
    base = w * EPW

    @pl.loop(0, EPW // CG)
    def _chunk(i):
        off = base + i * CG
        pltpu.sync_copy(row_hbm.at[pl.ds(off, CG)], idx_v)
        pltpu.async_copy(nodes_hbm.at[idx_v], rows_v, sem).wait()
        pltpu.sync_copy(rows_v, src_out.at[pl.ds(off, CG)])
        pltpu.sync_copy(col_hbm.at[pl.ds(off, CG)], idx_v)
        pltpu.async_copy(nodes_hbm.at[idx_v], rows_v, sem).wait()
        pltpu.sync_copy(rows_v, dst_out.at[pl.ds(off, CG)])


@functools.cache
def _gather():
    return pl.kernel(
        _gather_body,
        out_type=[jax.ShapeDtypeStruct((E_PAD, C), _f32)] * 2,
        mesh=plsc.VectorSubcoreMesh(core_axis_name="c", subcore_axis_name="s"),
        compiler_params=_SC_PARAMS,
        scratch_types=[
            pltpu.VMEM((CG,), jnp.int32),
            pltpu.VMEM((CG, C), _f32),
            pltpu.SemaphoreType.DMA,
        ],
    )


def _scatter_body(ma_hbm, mb_hbm, col_hbm, agga_hbm, aggb_hbm, idx_v, vals_v,
                  zbuf, acc):
    cid = lax.axis_index("c")
    sid = lax.axis_index("s")

    @pl.loop(0, 1024)
    def _zero_zbuf(r):
        zbuf[r, pl.ds(0, 16)] = jnp.zeros((16,), _f32)

    def _pass(m_hbm, lane0, agg_hbm, olane0):
        for ro, rn in _ZCH:
            pltpu.sync_copy(zbuf.at[pl.ds(0, rn)],
                            acc.at[pl.ds(sid * RPT + ro, rn)])
        plsc.subcore_barrier()

        @pl.loop(0, EPT // CS)
        def _chunk(i):
            off = sid * EPT + i * CS
            pltpu.sync_copy(col_hbm.at[pl.ds(off, CS)], idx_v)
            pltpu.sync_copy(m_hbm.at[pl.ds(off, CS), pl.ds(lane0, 16)],
                            vals_v)
            pltpu.sync_copy(vals_v, acc.at[idx_v], add=True)

        plsc.subcore_barrier()
        for ro, rn in _ZCH:
            pltpu.sync_copy(
                acc.at[pl.ds(sid * RPT + ro, rn)],
                agg_hbm.at[pl.ds(sid * RPT + ro, rn), pl.ds(olane0, 16)])
        plsc.subcore_barrier()

    @pl.when(cid == 0)
    def _core0():
        for g in range(4):
            _pass(ma_hbm, 16 * g, agga_hbm, 16 * g)

    @pl.when(cid == 1)
    def _core1():
        for g in range(4, 8):
            _pass(ma_hbm, 16 * g, agga_hbm, 16 * g)
        _pass(mb_hbm, 0, aggb_hbm, 0)


@functools.cache
def _scatter():
    return pl.kernel(
        _scatter_body,
        out_type=[jax.ShapeDtypeStruct((N_PAD, C), _f32)] * 2,
        mesh=plsc.VectorSubcoreMesh(core_axis_name="c", subcore_axis_name="s"),
        compiler_params=_SC_PARAMS,
        scratch_types=[
            pltpu.VMEM((CS,), jnp.int32),
            pltpu.VMEM((CS, 16), _f32),
            pltpu.VMEM((1024, 16), _f32),
            pltpu.VMEM_SHARED((N_PAD, 16), _f32),
        ],
    )


def _edge_body(src_ref, dst_ref, ep_ref, a1_ref, b1m_ref, w1s_ref, w2a_ref,
               c_ref, ma_ref, mb_ref):
    src = src_ref[...]
    dst = dst_ref[...]
    e0 = ep_ref[:, 3:4]
    e1 = ep_ref[:, 4:5]

    def row(i):
        return c_ref[i:i + 1, :]

    def scal(k):
        return c_ref[14:15, k:k + 1]

    z = (jnp.dot(src, a1_ref[...], preferred_element_type=_f32)
         + jnp.dot(dst, b1m_ref[...], preferred_element_type=_f32)
         + e0 * row(4) + e1 * row(5) + row(0))
    h = jnp.maximum(z, 0.0)
    d = 0.5 * (jnp.sum(h * row(1), axis=1, keepdims=True)
               + jnp.sum(src * row(2), axis=1, keepdims=True)
               + jnp.sum(dst * row(3), axis=1, keepdims=True)
               + e0 * scal(1) + e1 * scal(2) + scal(0))
    s = d * lax.rsqrt(d * d + 1e-5)
    e0n = e0 + scal(3) * s + scal(5)
    e1n = e1 - scal(4) * s + scal(6)

    zz = (jnp.dot(src, w1s_ref[...], preferred_element_type=_f32)
          + e0n * row(6) + e1n * row(7) + row(8))
    hh = jnp.maximum(zz, 0.0)
    y = src + jnp.dot(hh, w2a_ref[...], preferred_element_type=_f32) + row(9)
    y0 = e0n + jnp.sum(hh * row(12), axis=1, keepdims=True) + scal(7)
    y1 = e1n + jnp.sum(hh * row(13), axis=1, keepdims=True) + scal(8)
    tot = float(C + EC)
    mean = (jnp.sum(y, axis=1, keepdims=True) + y0 + y1) / tot
    var = (jnp.sum((y - mean) ** 2, axis=1, keepdims=True)
           + (y0 - mean) ** 2 + (y1 - mean) ** 2) / tot
    inv = lax.rsqrt(var + 1e-5)
    ma_ref[...] = (y - mean) * inv * row(10) + row(11)
    o0 = (y0 - mean) * inv * scal(9) + scal(11)
    o1 = (y1 - mean) * inv * scal(10) + scal(12)
    mb_ref[...] = jnp.concatenate(
        [o0, o1, jnp.ones_like(o0), e0n, e1n,
         jnp.zeros((EB, C - 5), _f32)], axis=1)


_edge_call = pl.pallas_call(
    _edge_body,
    grid=(E_PAD // EB,),
    in_specs=[
        pl.BlockSpec((EB, C), lambda i: (i, 0)),
        pl.BlockSpec((EB, C), lambda i: (i, 0)),
        pl.BlockSpec((EB, C), lambda i: (i, 0)),
        pl.BlockSpec((C, C), lambda i: (0, 0)),
        pl.BlockSpec((C, C), lambda i: (0, 0)),
        pl.BlockSpec((C, C), lambda i: (0, 0)),
        pl.BlockSpec((C, C), lambda i: (0, 0)),
        pl.BlockSpec((16, C), lambda i: (0, 0)),
    ],
    out_specs=[pl.BlockSpec((EB, C), lambda i: (i, 0))] * 2,
    out_shape=[jax.ShapeDtypeStruct((E_PAD, C), _f32)] * 2,
)


def _node_body(n_ref, aa_ref, ab_ref, w1a_ref, w1b_ref, w2_ref, rwa_ref,
               rwb_ref, c_ref, out_ref):
    nodes = n_ref[...]

    def row(i):
        return c_ref[i:i + 1, :]

    inv_d = 1.0 / jnp.maximum(ab_ref[:, 2:3], 1.0)
    aggc = aa_ref[...] * inv_d
    ae0 = ab_ref[:, 0:1] * inv_d
    ae1 = ab_ref[:, 1:2] * inv_d

    z = (jnp.dot(nodes, w1a_ref[...], preferred_element_type=_f32)
         + jnp.dot(aggc, w1b_ref[...], preferred_element_type=_f32)
         + ae0 * row(5) + ae1 * row(6) + row(0))
    hh = jnp.dot(jnp.maximum(z, 0.0), w2_ref[...],
                 preferred_element_type=_f32) + row(1)
    r = (jnp.dot(nodes, rwa_ref[...], preferred_element_type=_f32)
         + jnp.dot(aggc, rwb_ref[...], preferred_element_type=_f32)
         + ae0 * row(7) + ae1 * row(8) + row(2))
    y = r + hh
    mean = jnp.mean(y, axis=1, keepdims=True)
    var = jnp.mean((y - mean) ** 2, axis=1, keepdims=True)
    out_ref[...] = (y - mean) * lax.rsqrt(var + 1e-5) * row(3) + row(4) + nodes


_node_call = pl.pallas_call(
    _node_body,
    grid=(N_PAD // NB,),
    in_specs=[
        pl.BlockSpec((NB, C), lambda i: (i, 0)),
        pl.BlockSpec((NB, C), lambda i: (i, 0)),
        pl.BlockSpec((NB, C), lambda i: (i, 0)),
        pl.BlockSpec((C, C), lambda i: (0, 0)),
        pl.BlockSpec((C, C), lambda i: (0, 0)),
        pl.BlockSpec((C, C), lambda i: (0, 0)),
        pl.BlockSpec((C, C), lambda i: (0, 0)),
        pl.BlockSpec((C, C), lambda i: (0, 0)),
        pl.BlockSpec((16, C), lambda i: (0, 0)),
    ],
    out_specs=pl.BlockSpec((NB, C), lambda i: (i, 0)),
    out_shape=jax.ShapeDtypeStruct((N_PAD, C), _f32),
)


def kernel(x, edge_index, edge_attr, em_res_w, em_res_b, em_w1, em_b1, em_w2,
           em_b2, em_ln_g, em_ln_b, n1_w1, n1_b1, n1_w2, n1_b2, n1_ln_g,
           n1_ln_b, n2_res_w, n2_res_b, n2_w1, n2_b1, n2_w2, n2_b2, n2_ln_g,
           n2_ln_b):
    pad = N_NODES - N_INPUT
    nodes = jnp.zeros((N_PAD, C), _f32).at[pad:N_NODES].set(x)
    row_pad = jnp.concatenate(
        [edge_index[0].astype(jnp.int32),
         jnp.zeros((E_PAD - N_EDGES,), jnp.int32)])
    col_pad = jnp.concatenate(
        [edge_index[1].astype(jnp.int32),
         jnp.full((E_PAD - N_EDGES,), DUMP, jnp.int32)])
    ep = jnp.zeros((E_PAD, C), _f32).at[:N_EDGES, 3:5].set(edge_attr)

    for i in range(L):
        wd = em_w2[i][:, 0] - em_w2[i][:, 1]
        rd = em_res_w[i][:, 0] - em_res_w[i][:, 1]
        cd = (em_b2[i][0] - em_b2[i][1]) + (em_res_b[i][0] - em_res_b[i][1])
        ce = jnp.zeros((16, C), _f32)
        ce = ce.at[0].set(em_b1[i])
        ce = ce.at[1].set(wd)
        ce = ce.at[2].set(rd[:C])
        ce = ce.at[3].set(rd[C:2 * C])
        ce = ce.at[4].set(em_w1[i][2 * C])
        ce = ce.at[5].set(em_w1[i][2 * C + 1])
        ce = ce.at[6].set(n1_w1[i][C])
        ce = ce.at[7].set(n1_w1[i][C + 1])
        ce = ce.at[8].set(n1_b1[i])
        ce = ce.at[9].set(n1_b2[i][:C])
        ce = ce.at[10].set(n1_ln_g[i][:C])
        ce = ce.at[11].set(n1_ln_b[i][:C])
        ce = ce.at[12].set(n1_w2[i][:, C])
        ce = ce.at[13].set(n1_w2[i][:, C + 1])
        sc = jnp.stack([
            cd, rd[2 * C], rd[2 * C + 1], em_ln_g[i][0], em_ln_g[i][1],
            em_ln_b[i][0], em_ln_b[i][1], n1_b2[i][C], n1_b2[i][C + 1],
            n1_ln_g[i][C], n1_ln_g[i][C + 1], n1_ln_b[i][C],
            n1_ln_b[i][C + 1],
        ])
        ce = ce.at[14, :13].set(sc)

        cn = jnp.zeros((16, C), _f32)
        cn = cn.at[0].set(n2_b1[i])
        cn = cn.at[1].set(n2_b2[i])
        cn = cn.at[2].set(n2_res_b[i])
        cn = cn.at[3].set(n2_ln_g[i])
        cn = cn.at[4].set(n2_ln_b[i])
        cn = cn.at[5].set(n2_w1[i][2 * C])
        cn = cn.at[6].set(n2_w1[i][2 * C + 1])
        cn = cn.at[7].set(n2_res_w[i][2 * C])
        cn = cn.at[8].set(n2_res_w[i][2 * C + 1])

        src, dst = _gather()(nodes, row_pad, col_pad)
        ma, mb = _edge_call(src, dst, ep, em_w1[i][:C], em_w1[i][C:2 * C],
                            n1_w1[i][:C], n1_w2[i][:, :C], ce)
        agga, aggb = _scatter()(ma, mb, col_pad)
        nodes = _node_call(nodes, agga, aggb, n2_w1[i][:C],
                           n2_w1[i][C:2 * C], n2_w2[i], n2_res_w[i][:C],
                           n2_res_w[i][C:2 * C], cn)
        ep = mb

    return nodes[pad:N_NODES]


# speedup vs baseline: 2.5784x; 1.2252x over previous
import functools

import jax
import jax.numpy as jnp
from jax import lax
from jax.experimental import pallas as pl
from jax.experimental.pallas import tpu as pltpu
from jax.experimental.pallas import tpu_sc as plsc

L = 4
C = 128
EC = 2
H = 128
N_NODES = 48008
N_INPUT = 41162
N_EDGES = 432072

N_PAD = 48128
E_PAD = 442368
DUMP = N_NODES
EB = 1024
NB = 512
NW = 32
EPW = E_PAD // NW
CG = 192
CS = 512
EPT = E_PAD // 16
RPT = N_PAD // 16
_ZCH = ((0, 1024), (1024, 1024), (2048, 960))

_f32 = jnp.float32
_bf16 = jnp.bfloat16
_SC_PARAMS = pltpu.CompilerParams(use_tc_tiling_on_sc=False)


def _gather_body(nodes_hbm, row_hbm, col_hbm, src_out, dst_out, i0, i1, i2,
                 i3, r0, r1, r2, r3, s0, s1, s2, s3):
    w = lax.axis_index("s") * 2 + lax.axis_index("c")
    base = w * EPW
    nch = EPW // CG
    idxs = (i0, i1, i2, i3)
    rows = (r0, r1, r2, r3)
    sems = (s0, s1, s2, s3)
    srcs = (row_hbm, col_hbm)
    outs = (src_out, dst_out)

    def issue(t, i):
        b = 2 * t + (i & 1)
        off = base + i * CG
        pltpu.sync_copy(srcs[t].at[pl.ds(off, CG)], idxs[b])
        return pltpu.async_copy(nodes_hbm.at[idxs[b]], rows[b], sems[b])

    def drain(t, i, pend):
        pend.wait()
        off = base + i * CG
        pltpu.sync_copy(rows[2 * t + (i & 1)], outs[t].at[pl.ds(off, CG)])

    pend = [issue(0, 0), issue(1, 0)]
    for i in range(1, nch):
        nxt = [issue(0, i), issue(1, i)]
        for t in (0, 1):
            drain(t, i - 1, pend[t])
        pend = nxt
    for t in (0, 1):
        drain(t, nch - 1, pend[t])


@functools.cache
def _gather():
    return pl.kernel(
        _gather_body,
        out_type=[jax.ShapeDtypeStruct((E_PAD, C), _f32)] * 2,
        mesh=plsc.VectorSubcoreMesh(core_axis_name="c", subcore_axis_name="s"),
        compiler_params=_SC_PARAMS,
        scratch_types=[pltpu.VMEM((CG,), jnp.int32)] * 4
        + [pltpu.VMEM((CG, C), _f32)] * 4
        + [pltpu.SemaphoreType.DMA] * 4,
    )


def _scatter_body(ma_hbm, mb_hbm, col_hbm, agga_hbm, aggb_hbm, idx_v, v0, v1,
                  zbuf, acc, sm0, sm1):
    cid = lax.axis_index("c")
    sid = lax.axis_index("s")
    nch = EPT // CS

    @pl.loop(0, 1024)
    def _zero_zbuf(r):
        zbuf[r, pl.ds(0, 16)] = jnp.zeros((16,), _f32)

    pltpu.sync_copy(col_hbm.at[pl.ds(sid * EPT, EPT)], idx_v)
    vbufs = (v0, v1)
    sems = (sm0, sm1)

    def _pass(m_hbm, lane0, agg_hbm, olane0, wide):
        for ro, rn in _ZCH:
            pltpu.sync_copy(zbuf.at[pl.ds(0, rn)],
                            acc.at[pl.ds(sid * RPT + ro, rn)])
        plsc.subcore_barrier()

        def load(i):
            off = sid * EPT + i * CS
            return pltpu.async_copy(
                m_hbm.at[pl.ds(off, CS), pl.ds(lane0, 16)], vbufs[i & 1],
                sems[i & 1])

        pend = load(0)
        for i in range(nch):
            nxt = load(i + 1) if i + 1 < nch else None
            pend.wait()
            pltpu.sync_copy(vbufs[i & 1], acc.at[idx_v.at[pl.ds(i * CS, CS)]],
                            add=True)
            pend = nxt

        plsc.subcore_barrier()
        for ro, rn in _ZCH:
            s = pl.ds(sid * RPT + ro, rn)
            if wide:
                pltpu.sync_copy(acc.at[s], agg_hbm.at[s, pl.ds(olane0, 16)])
            else:
                pltpu.sync_copy(acc.at[s], agg_hbm.at[s])
        plsc.subcore_barrier()

    @pl.when(cid == 0)
    def _core0():
        for g in range(4):
            _pass(ma_hbm, 16 * g, agga_hbm, 16 * g, True)

    @pl.when(cid == 1)
    def _core1():
        for g in range(4, 8):
            _pass(ma_hbm, 16 * g, agga_hbm, 16 * g, True)
        _pass(mb_hbm, 0, aggb_hbm, 0, None)


@functools.cache
def _scatter():
    return pl.kernel(
        _scatter_body,
        out_type=[jax.ShapeDtypeStruct((N_PAD, C), _f32),
                  jax.ShapeDtypeStruct((N_PAD, 16), _f32)],
        mesh=plsc.VectorSubcoreMesh(core_axis_name="c", subcore_axis_name="s"),
        compiler_params=_SC_PARAMS,
        scratch_types=[
            pltpu.VMEM((EPT,), jnp.int32),
            pltpu.VMEM((CS, 16), _f32),
            pltpu.VMEM((CS, 16), _f32),
            pltpu.VMEM((1024, 16), _f32),
            pltpu.VMEM_SHARED((N_PAD, 16), _f32),
            pltpu.SemaphoreType.DMA,
            pltpu.SemaphoreType.DMA,
        ],
    )


def _edge_body(src_ref, dst_ref, ep_ref, a1_ref, b1m_ref, w1s_ref, w2a_ref,
               c_ref, ma_ref, mb_ref):
    src = src_ref[...]
    dst = dst_ref[...]
    srcf = src
    e0 = ep_ref[:, 3:4]
    e1 = ep_ref[:, 4:5]

    def row(i):
        return c_ref[i:i + 1, :]

    def scal(k):
        return c_ref[14:15, k:k + 1]

    dstf = dst

    z = (jnp.dot(src, a1_ref[...], preferred_element_type=_f32)
         + jnp.dot(dst, b1m_ref[...], preferred_element_type=_f32)
         + e0 * row(4) + e1 * row(5) + row(0))
    h = jnp.maximum(z, 0.0)
    d_pre = h * row(1) + srcf * row(2) + dstf * row(3)
    d = 0.5 * (jnp.sum(d_pre, axis=1, keepdims=True)
               + e0 * scal(1) + e1 * scal(2) + scal(0))
    s = d * lax.rsqrt(d * d + 1e-5)
    e0n = e0 + scal(3) * s + scal(5)
    e1n = e1 - scal(4) * s + scal(6)

    zz = (jnp.dot(src, w1s_ref[...], preferred_element_type=_f32)
          + e0n * row(6) + e1n * row(7) + row(8))
    hhf = jnp.maximum(zz, 0.0)
    hh = hhf
    y = srcf + jnp.dot(hh, w2a_ref[...], preferred_element_type=_f32) + row(9)
    y0 = e0n + jnp.sum(hhf * row(12), axis=1, keepdims=True) + scal(7)
    y1 = e1n + jnp.sum(hhf * row(13), axis=1, keepdims=True) + scal(8)
    tot = float(C + EC)
    sy = jnp.sum(y + y * y, axis=1, keepdims=True)
    sy2 = jnp.sum((y - 1.0) * y, axis=1, keepdims=True)
    mean = (0.5 * (sy - sy2) + y0 + y1) / tot
    var = (0.5 * (sy + sy2) + y0 * y0 + y1 * y1) / tot - mean * mean
    inv = lax.rsqrt(var + 1e-5)
    ma_ref[...] = (y - mean) * inv * row(10) + row(11)
    o0 = (y0 - mean) * inv * scal(9) + scal(11)
    o1 = (y1 - mean) * inv * scal(10) + scal(12)
    mb_ref[...] = jnp.concatenate(
        [o0, o1, jnp.ones_like(o0), e0n, e1n,
         jnp.zeros((EB, 27), _f32)], axis=1)


_edge_call = pl.pallas_call(
    _edge_body,
    grid=(E_PAD // EB,),
    in_specs=[
        pl.BlockSpec((EB, C), lambda i: (i, 0)),
        pl.BlockSpec((EB, C), lambda i: (i, 0)),
        pl.BlockSpec((EB, 32), lambda i: (i, 0)),
        pl.BlockSpec((C, C), lambda i: (0, 0)),
        pl.BlockSpec((C, C), lambda i: (0, 0)),
        pl.BlockSpec((C, C), lambda i: (0, 0)),
        pl.BlockSpec((C, C), lambda i: (0, 0)),
        pl.BlockSpec((16, C), lambda i: (0, 0)),
    ],
    out_specs=[pl.BlockSpec((EB, C), lambda i: (i, 0)),
               pl.BlockSpec((EB, 32), lambda i: (i, 0))],
    out_shape=[jax.ShapeDtypeStruct((E_PAD, C), _f32),
               jax.ShapeDtypeStruct((E_PAD, 32), _f32)],
)


def _node_body(n_ref, aa_ref, ab_ref, w1a_ref, w1b_ref, w2_ref, rwa_ref,
               rwb_ref, c_ref, out_ref, outb_ref):
    nodes = n_ref[...]

    def row(i):
        return c_ref[i:i + 1, :]

    inv_d = 1.0 / jnp.maximum(ab_ref[:, 2:3], 1.0)
    aggc = aa_ref[...] * inv_d
    ae0 = ab_ref[:, 0:1] * inv_d
    ae1 = ab_ref[:, 1:2] * inv_d

    z = (jnp.dot(nodes, w1a_ref[...], preferred_element_type=_f32)
         + jnp.dot(aggc, w1b_ref[...], preferred_element_type=_f32)
         + ae0 * row(5) + ae1 * row(6) + row(0))
    hh = jnp.dot(jnp.maximum(z, 0.0), w2_ref[...],
                 preferred_element_type=_f32) + row(1)
    r = (jnp.dot(nodes, rwa_ref[...], preferred_element_type=_f32)
         + jnp.dot(aggc, rwb_ref[...], preferred_element_type=_f32)
         + ae0 * row(7) + ae1 * row(8) + row(2))
    y = r + hh
    mean = jnp.mean(y, axis=1, keepdims=True)
    var = jnp.mean((y - mean) ** 2, axis=1, keepdims=True)
    out = (y - mean) * lax.rsqrt(var + 1e-5) * row(3) + row(4) + nodes
    out_ref[...] = out
    outb_ref[...] = out.astype(_bf16)


_node_call = pl.pallas_call(
    _node_body,
    grid=(N_PAD // NB,),
    in_specs=[
        pl.BlockSpec((NB, C), lambda i: (i, 0)),
        pl.BlockSpec((NB, C), lambda i: (i, 0)),
        pl.BlockSpec((NB, 16), lambda i: (i, 0)),
        pl.BlockSpec((C, C), lambda i: (0, 0)),
        pl.BlockSpec((C, C), lambda i: (0, 0)),
        pl.BlockSpec((C, C), lambda i: (0, 0)),
        pl.BlockSpec((C, C), lambda i: (0, 0)),
        pl.BlockSpec((C, C), lambda i: (0, 0)),
        pl.BlockSpec((16, C), lambda i: (0, 0)),
    ],
    out_specs=[pl.BlockSpec((NB, C), lambda i: (i, 0))] * 2,
    out_shape=[jax.ShapeDtypeStruct((N_PAD, C), _f32),
               jax.ShapeDtypeStruct((N_PAD, C), _bf16)],
)


def kernel(x, edge_index, edge_attr, em_res_w, em_res_b, em_w1, em_b1, em_w2,
           em_b2, em_ln_g, em_ln_b, n1_w1, n1_b1, n1_w2, n1_b2, n1_ln_g,
           n1_ln_b, n2_res_w, n2_res_b, n2_w1, n2_b1, n2_w2, n2_b2, n2_ln_g,
           n2_ln_b):
    pad = N_NODES - N_INPUT
    nodes = jnp.zeros((N_PAD, C), _f32).at[pad:N_NODES].set(x)
    nodes_b = nodes
    row_pad = jnp.concatenate(
        [edge_index[0].astype(jnp.int32),
         jnp.zeros((E_PAD - N_EDGES,), jnp.int32)])
    col_pad = jnp.concatenate(
        [edge_index[1].astype(jnp.int32),
         jnp.full((E_PAD - N_EDGES,), DUMP, jnp.int32)])
    ep = jnp.zeros((E_PAD, 32), _f32).at[:N_EDGES, 3:5].set(edge_attr)

    for i in range(L):
        wd = em_w2[i][:, 0] - em_w2[i][:, 1]
        rd = em_res_w[i][:, 0] - em_res_w[i][:, 1]
        cd = (em_b2[i][0] - em_b2[i][1]) + (em_res_b[i][0] - em_res_b[i][1])
        ce = jnp.zeros((16, C), _f32)
        ce = ce.at[0].set(em_b1[i])
        ce = ce.at[1].set(wd)
        ce = ce.at[2].set(rd[:C])
        ce = ce.at[3].set(rd[C:2 * C])
        ce = ce.at[4].set(em_w1[i][2 * C])
        ce = ce.at[5].set(em_w1[i][2 * C + 1])
        ce = ce.at[6].set(n1_w1[i][C])
        ce = ce.at[7].set(n1_w1[i][C + 1])
        ce = ce.at[8].set(n1_b1[i])
        ce = ce.at[9].set(n1_b2[i][:C])
        ce = ce.at[10].set(n1_ln_g[i][:C])
        ce = ce.at[11].set(n1_ln_b[i][:C])
        ce = ce.at[12].set(n1_w2[i][:, C])
        ce = ce.at[13].set(n1_w2[i][:, C + 1])
        sc = jnp.stack([
            cd, rd[2 * C], rd[2 * C + 1], em_ln_g[i][0], em_ln_g[i][1],
            em_ln_b[i][0], em_ln_b[i][1], n1_b2[i][C], n1_b2[i][C + 1],
            n1_ln_g[i][C], n1_ln_g[i][C + 1], n1_ln_b[i][C],
            n1_ln_b[i][C + 1],
        ])
        ce = ce.at[14, :13].set(sc)

        cn = jnp.zeros((16, C), _f32)
        cn = cn.at[0].set(n2_b1[i])
        cn = cn.at[1].set(n2_b2[i])
        cn = cn.at[2].set(n2_res_b[i])
        cn = cn.at[3].set(n2_ln_g[i])
        cn = cn.at[4].set(n2_ln_b[i])
        cn = cn.at[5].set(n2_w1[i][2 * C])
        cn = cn.at[6].set(n2_w1[i][2 * C + 1])
        cn = cn.at[7].set(n2_res_w[i][2 * C])
        cn = cn.at[8].set(n2_res_w[i][2 * C + 1])

        src, dst = _gather()(nodes, row_pad, col_pad)
        ma, mb = _edge_call(src, dst, ep, em_w1[i][:C], em_w1[i][C:2 * C],
                            n1_w1[i][:C], n1_w2[i][:, :C], ce)
        agga, aggb = _scatter()(ma, mb, col_pad)
        nodes, nodes_b = _node_call(nodes, agga, aggb, n2_w1[i][:C],
                                    n2_w1[i][C:2 * C], n2_w2[i],
                                    n2_res_w[i][:C], n2_res_w[i][C:2 * C], cn)
        ep = mb

    return nodes[pad:N_NODES]


# speedup vs baseline: 2.6050x; 1.0103x over previous
import functools

import jax
import jax.numpy as jnp
from jax import lax
from jax.experimental import pallas as pl
from jax.experimental.pallas import tpu as pltpu
from jax.experimental.pallas import tpu_sc as plsc

L = 4
C = 128
EC = 2
H = 128
N_NODES = 48008
N_INPUT = 41162
N_EDGES = 432072

N_PAD = 48128
E_PAD = 442368
DUMP = N_NODES
EB = 1024
NB = 512
NW = 32
EPW = E_PAD // NW
CG = 192
CS = 512
EPT = E_PAD // 16
RPT = N_PAD // 16
_ZCH = ((0, 1024), (1024, 1024), (2048, 960))

_f32 = jnp.float32
_bf16 = jnp.bfloat16
_SC_PARAMS = pltpu.CompilerParams(use_tc_tiling_on_sc=False)


def _gather_body(nodes_hbm, row_hbm, col_hbm, src_out, dst_out, ia, ib, r0,
                 r1, r2, r3, s0, s1, s2, s3):
    w = lax.axis_index("s") * 2 + lax.axis_index("c")
    base = w * EPW
    nch = EPW // CG
    rows = (r0, r1, r2, r3)
    sems = (s0, s1, s2, s3)
    idxs = (ia, ib)
    outs = (src_out, dst_out)
    pltpu.sync_copy(row_hbm.at[pl.ds(base, EPW)], ia)
    pltpu.sync_copy(col_hbm.at[pl.ds(base, EPW)], ib)

    def issue(t, i):
        b = 2 * t + (i & 1)
        return pltpu.async_copy(
            nodes_hbm.at[idxs[t].at[pl.ds(i * CG, CG)]], rows[b], sems[b])

    def drain(t, i, pend):
        pend.wait()
        off = base + i * CG
        pltpu.sync_copy(rows[2 * t + (i & 1)], outs[t].at[pl.ds(off, CG)])

    pend = [issue(0, 0), issue(1, 0)]
    for i in range(1, nch):
        nxt = [issue(0, i), issue(1, i)]
        for t in (0, 1):
            drain(t, i - 1, pend[t])
        pend = nxt
    for t in (0, 1):
        drain(t, nch - 1, pend[t])


@functools.cache
def _gather():
    return pl.kernel(
        _gather_body,
        out_type=[jax.ShapeDtypeStruct((E_PAD, C), _f32)] * 2,
        mesh=plsc.VectorSubcoreMesh(core_axis_name="c", subcore_axis_name="s"),
        compiler_params=_SC_PARAMS,
        scratch_types=[pltpu.VMEM((EPW,), jnp.int32)] * 2
        + [pltpu.VMEM((CG, C), _f32)] * 4
        + [pltpu.SemaphoreType.DMA] * 4,
    )


def _scatter_body(ma_hbm, mb_hbm, col_hbm, agga_hbm, aggb_hbm, idx_v, v0, v1,
                  zbuf, acc, sm0, sm1):
    cid = lax.axis_index("c")
    sid = lax.axis_index("s")
    nch = EPT // CS

    @pl.loop(0, 1024)
    def _zero_zbuf(r):
        zbuf[r, pl.ds(0, 16)] = jnp.zeros((16,), _f32)

    pltpu.sync_copy(col_hbm.at[pl.ds(sid * EPT, EPT)], idx_v)
    vbufs = (v0, v1)
    sems = (sm0, sm1)

    def _pass(m_hbm, lane0, agg_hbm, olane0, wide):
        for ro, rn in _ZCH:
            pltpu.sync_copy(zbuf.at[pl.ds(0, rn)],
                            acc.at[pl.ds(sid * RPT + ro, rn)])
        plsc.subcore_barrier()

        def load(i):
            off = sid * EPT + i * CS
            return pltpu.async_copy(
                m_hbm.at[pl.ds(off, CS), pl.ds(lane0, 16)], vbufs[i & 1],
                sems[i & 1])

        pend = load(0)
        for i in range(nch):
            nxt = load(i + 1) if i + 1 < nch else None
            pend.wait()
            pltpu.sync_copy(vbufs[i & 1], acc.at[idx_v.at[pl.ds(i * CS, CS)]],
                            add=True)
            pend = nxt

        plsc.subcore_barrier()
        for ro, rn in _ZCH:
            s = pl.ds(sid * RPT + ro, rn)
            if wide:
                pltpu.sync_copy(acc.at[s], agg_hbm.at[s, pl.ds(olane0, 16)])
            else:
                pltpu.sync_copy(acc.at[s], agg_hbm.at[s])
        plsc.subcore_barrier()

    @pl.when(cid == 0)
    def _core0():
        for g in range(4):
            _pass(ma_hbm, 16 * g, agga_hbm, 16 * g, True)

    @pl.when(cid == 1)
    def _core1():
        for g in range(4, 8):
            _pass(ma_hbm, 16 * g, agga_hbm, 16 * g, True)
        _pass(mb_hbm, 0, aggb_hbm, 0, None)


@functools.cache
def _scatter():
    return pl.kernel(
        _scatter_body,
        out_type=[jax.ShapeDtypeStruct((N_PAD, C), _f32),
                  jax.ShapeDtypeStruct((N_PAD, 16), _f32)],
        mesh=plsc.VectorSubcoreMesh(core_axis_name="c", subcore_axis_name="s"),
        compiler_params=_SC_PARAMS,
        scratch_types=[
            pltpu.VMEM((EPT,), jnp.int32),
            pltpu.VMEM((CS, 16), _f32),
            pltpu.VMEM((CS, 16), _f32),
            pltpu.VMEM((1024, 16), _f32),
            pltpu.VMEM_SHARED((N_PAD, 16), _f32),
            pltpu.SemaphoreType.DMA,
            pltpu.SemaphoreType.DMA,
        ],
    )


def _edge_body(src_ref, dst_ref, ep_ref, a1_ref, b1m_ref, w1s_ref, w2a_ref,
               c_ref, ma_ref, mb_ref):
    src = src_ref[...]
    dst = dst_ref[...]
    srcf = src
    e0 = ep_ref[:, 3:4]
    e1 = ep_ref[:, 4:5]

    def row(i):
        return c_ref[i:i + 1, :]

    def scal(k):
        return c_ref[14:15, k:k + 1]

    dstf = dst

    z = (jnp.dot(src, a1_ref[...], preferred_element_type=_f32)
         + jnp.dot(dst, b1m_ref[...], preferred_element_type=_f32)
         + e0 * row(4) + e1 * row(5) + row(0))
    h = jnp.maximum(z, 0.0)
    d_pre = h * row(1) + srcf * row(2) + dstf * row(3)
    d = 0.5 * (jnp.sum(d_pre, axis=1, keepdims=True)
               + e0 * scal(1) + e1 * scal(2) + scal(0))
    s = d * lax.rsqrt(d * d + 1e-5)
    e0n = e0 + scal(3) * s + scal(5)
    e1n = e1 - scal(4) * s + scal(6)

    zz = (jnp.dot(src, w1s_ref[...], preferred_element_type=_f32)
          + e0n * row(6) + e1n * row(7) + row(8))
    hhf = jnp.maximum(zz, 0.0)
    hh = hhf
    y = srcf + jnp.dot(hh, w2a_ref[...], preferred_element_type=_f32) + row(9)
    y0 = e0n + jnp.sum(hhf * row(12), axis=1, keepdims=True) + scal(7)
    y1 = e1n + jnp.sum(hhf * row(13), axis=1, keepdims=True) + scal(8)
    tot = float(C + EC)
    sy = jnp.sum(y + y * y, axis=1, keepdims=True)
    sy2 = jnp.sum((y - 1.0) * y, axis=1, keepdims=True)
    mean = (0.5 * (sy - sy2) + y0 + y1) / tot
    var = (0.5 * (sy + sy2) + y0 * y0 + y1 * y1) / tot - mean * mean
    inv = lax.rsqrt(var + 1e-5)
    ma_ref[...] = (y - mean) * inv * row(10) + row(11)
    o0 = (y0 - mean) * inv * scal(9) + scal(11)
    o1 = (y1 - mean) * inv * scal(10) + scal(12)
    mb_ref[...] = jnp.concatenate(
        [o0, o1, jnp.ones_like(o0), e0n, e1n,
         jnp.zeros((EB, 27), _f32)], axis=1)


_edge_call = pl.pallas_call(
    _edge_body,
    grid=(E_PAD // EB,),
    in_specs=[
        pl.BlockSpec((EB, C), lambda i: (i, 0)),
        pl.BlockSpec((EB, C), lambda i: (i, 0)),
        pl.BlockSpec((EB, 32), lambda i: (i, 0)),
        pl.BlockSpec((C, C), lambda i: (0, 0)),
        pl.BlockSpec((C, C), lambda i: (0, 0)),
        pl.BlockSpec((C, C), lambda i: (0, 0)),
        pl.BlockSpec((C, C), lambda i: (0, 0)),
        pl.BlockSpec((16, C), lambda i: (0, 0)),
    ],
    out_specs=[pl.BlockSpec((EB, C), lambda i: (i, 0)),
               pl.BlockSpec((EB, 32), lambda i: (i, 0))],
    out_shape=[jax.ShapeDtypeStruct((E_PAD, C), _f32),
               jax.ShapeDtypeStruct((E_PAD, 32), _f32)],
)


def _node_body(n_ref, aa_ref, ab_ref, w1a_ref, w1b_ref, w2_ref, rwa_ref,
               rwb_ref, c_ref, out_ref, outb_ref):
    nodes = n_ref[...]

    def row(i):
        return c_ref[i:i + 1, :]

    inv_d = 1.0 / jnp.maximum(ab_ref[:, 2:3], 1.0)
    aggc = aa_ref[...] * inv_d
    ae0 = ab_ref[:, 0:1] * inv_d
    ae1 = ab_ref[:, 1:2] * inv_d

    z = (jnp.dot(nodes, w1a_ref[...], preferred_element_type=_f32)
         + jnp.dot(aggc, w1b_ref[...], preferred_element_type=_f32)
         + ae0 * row(5) + ae1 * row(6) + row(0))
    hh = jnp.dot(jnp.maximum(z, 0.0), w2_ref[...],
                 preferred_element_type=_f32) + row(1)
    r = (jnp.dot(nodes, rwa_ref[...], preferred_element_type=_f32)
         + jnp.dot(aggc, rwb_ref[...], preferred_element_type=_f32)
         + ae0 * row(7) + ae1 * row(8) + row(2))
    y = r + hh
    mean = jnp.mean(y, axis=1, keepdims=True)
    var = jnp.mean((y - mean) ** 2, axis=1, keepdims=True)
    out = (y - mean) * lax.rsqrt(var + 1e-5) * row(3) + row(4) + nodes
    out_ref[...] = out
    outb_ref[...] = out.astype(_bf16)


_node_call = pl.pallas_call(
    _node_body,
    grid=(N_PAD // NB,),
    in_specs=[
        pl.BlockSpec((NB, C), lambda i: (i, 0)),
        pl.BlockSpec((NB, C), lambda i: (i, 0)),
        pl.BlockSpec((NB, 16), lambda i: (i, 0)),
        pl.BlockSpec((C, C), lambda i: (0, 0)),
        pl.BlockSpec((C, C), lambda i: (0, 0)),
        pl.BlockSpec((C, C), lambda i: (0, 0)),
        pl.BlockSpec((C, C), lambda i: (0, 0)),
        pl.BlockSpec((C, C), lambda i: (0, 0)),
        pl.BlockSpec((16, C), lambda i: (0, 0)),
    ],
    out_specs=[pl.BlockSpec((NB, C), lambda i: (i, 0))] * 2,
    out_shape=[jax.ShapeDtypeStruct((N_PAD, C), _f32),
               jax.ShapeDtypeStruct((N_PAD, C), _bf16)],
)


def kernel(x, edge_index, edge_attr, em_res_w, em_res_b, em_w1, em_b1, em_w2,
           em_b2, em_ln_g, em_ln_b, n1_w1, n1_b1, n1_w2, n1_b2, n1_ln_g,
           n1_ln_b, n2_res_w, n2_res_b, n2_w1, n2_b1, n2_w2, n2_b2, n2_ln_g,
           n2_ln_b):
    pad = N_NODES - N_INPUT
    nodes = jnp.zeros((N_PAD, C), _f32).at[pad:N_NODES].set(x)
    nodes_b = nodes
    row_pad = jnp.concatenate(
        [edge_index[0].astype(jnp.int32),
         jnp.zeros((E_PAD - N_EDGES,), jnp.int32)])
    col_pad = jnp.concatenate(
        [edge_index[1].astype(jnp.int32),
         jnp.full((E_PAD - N_EDGES,), DUMP, jnp.int32)])
    ep = jnp.zeros((E_PAD, 32), _f32).at[:N_EDGES, 3:5].set(edge_attr)

    for i in range(L):
        wd = em_w2[i][:, 0] - em_w2[i][:, 1]
        rd = em_res_w[i][:, 0] - em_res_w[i][:, 1]
        cd = (em_b2[i][0] - em_b2[i][1]) + (em_res_b[i][0] - em_res_b[i][1])
        ce = jnp.zeros((16, C), _f32)
        ce = ce.at[0].set(em_b1[i])
        ce = ce.at[1].set(wd)
        ce = ce.at[2].set(rd[:C])
        ce = ce.at[3].set(rd[C:2 * C])
        ce = ce.at[4].set(em_w1[i][2 * C])
        ce = ce.at[5].set(em_w1[i][2 * C + 1])
        ce = ce.at[6].set(n1_w1[i][C])
        ce = ce.at[7].set(n1_w1[i][C + 1])
        ce = ce.at[8].set(n1_b1[i])
        ce = ce.at[9].set(n1_b2[i][:C])
        ce = ce.at[10].set(n1_ln_g[i][:C])
        ce = ce.at[11].set(n1_ln_b[i][:C])
        ce = ce.at[12].set(n1_w2[i][:, C])
        ce = ce.at[13].set(n1_w2[i][:, C + 1])
        sc = jnp.stack([
            cd, rd[2 * C], rd[2 * C + 1], em_ln_g[i][0], em_ln_g[i][1],
            em_ln_b[i][0], em_ln_b[i][1], n1_b2[i][C], n1_b2[i][C + 1],
            n1_ln_g[i][C], n1_ln_g[i][C + 1], n1_ln_b[i][C],
            n1_ln_b[i][C + 1],
        ])
        ce = ce.at[14, :13].set(sc)

        cn = jnp.zeros((16, C), _f32)
        cn = cn.at[0].set(n2_b1[i])
        cn = cn.at[1].set(n2_b2[i])
        cn = cn.at[2].set(n2_res_b[i])
        cn = cn.at[3].set(n2_ln_g[i])
        cn = cn.at[4].set(n2_ln_b[i])
        cn = cn.at[5].set(n2_w1[i][2 * C])
        cn = cn.at[6].set(n2_w1[i][2 * C + 1])
        cn = cn.at[7].set(n2_res_w[i][2 * C])
        cn = cn.at[8].set(n2_res_w[i][2 * C + 1])

        src, dst = _gather()(nodes, row_pad, col_pad)
        ma, mb = _edge_call(src, dst, ep, em_w1[i][:C], em_w1[i][C:2 * C],
                            n1_w1[i][:C], n1_w2[i][:, :C], ce)
        agga, aggb = _scatter()(ma, mb, col_pad)
        nodes, nodes_b = _node_call(nodes, agga, aggb, n2_w1[i][:C],
                                    n2_w1[i][C:2 * C], n2_w2[i],
                                    n2_res_w[i][:C], n2_res_w[i][C:2 * C], cn)
        ep = mb

    return nodes[pad:N_NODES]


# speedup vs baseline: 2.6209x; 1.0061x over previous
import functools

import jax
import jax.numpy as jnp
from jax import lax
from jax.experimental import pallas as pl
from jax.experimental.pallas import tpu as pltpu
from jax.experimental.pallas import tpu_sc as plsc

L = 4
C = 128
EC = 2
H = 128
N_NODES = 48008
N_INPUT = 41162
N_EDGES = 432072

N_PAD = 48128
E_PAD = 442368
DUMP = N_NODES
EB = 1024
NB = 512
NW = 32
EPW = E_PAD // NW
CG = 192
CS = 1024
EPT = E_PAD // 16
RPT = N_PAD // 16
_ZCH = ((0, 1024), (1024, 1024), (2048, 960))

_f32 = jnp.float32
_bf16 = jnp.bfloat16
_SC_PARAMS = pltpu.CompilerParams(use_tc_tiling_on_sc=False)


def _gather_body(nodes_hbm, row_hbm, col_hbm, src_out, dst_out, ia, ib, r0,
                 r1, r2, r3, s0, s1, s2, s3):
    w = lax.axis_index("s") * 2 + lax.axis_index("c")
    base = w * EPW
    nch = EPW // CG
    rows = (r0, r1, r2, r3)
    sems = (s0, s1, s2, s3)
    idxs = (ia, ib)
    outs = (src_out, dst_out)
    pltpu.sync_copy(row_hbm.at[pl.ds(base, EPW)], ia)
    pltpu.sync_copy(col_hbm.at[pl.ds(base, EPW)], ib)

    def issue(t, i):
        b = 2 * t + (i & 1)
        return pltpu.async_copy(
            nodes_hbm.at[idxs[t].at[pl.ds(i * CG, CG)]], rows[b], sems[b])

    def drain(t, i, pend):
        pend.wait()
        off = base + i * CG
        pltpu.sync_copy(rows[2 * t + (i & 1)], outs[t].at[pl.ds(off, CG)])

    pend = [issue(0, 0), issue(1, 0)]
    for i in range(1, nch):
        nxt = [issue(0, i), issue(1, i)]
        for t in (0, 1):
            drain(t, i - 1, pend[t])
        pend = nxt
    for t in (0, 1):
        drain(t, nch - 1, pend[t])


@functools.cache
def _gather():
    return pl.kernel(
        _gather_body,
        out_type=[jax.ShapeDtypeStruct((E_PAD, C), _f32)] * 2,
        mesh=plsc.VectorSubcoreMesh(core_axis_name="c", subcore_axis_name="s"),
        compiler_params=_SC_PARAMS,
        scratch_types=[pltpu.VMEM((EPW,), jnp.int32)] * 2
        + [pltpu.VMEM((CG, C), _f32)] * 4
        + [pltpu.SemaphoreType.DMA] * 4,
    )


def _scatter_body(ma_hbm, mb_hbm, col_hbm, agga_hbm, aggb_hbm, idx_v, v0, v1,
                  zbuf, acc, sm0, sm1):
    cid = lax.axis_index("c")
    sid = lax.axis_index("s")
    nch = EPT // CS

    @pl.loop(0, 1024)
    def _zero_zbuf(r):
        zbuf[r, pl.ds(0, 16)] = jnp.zeros((16,), _f32)

    pltpu.sync_copy(col_hbm.at[pl.ds(sid * EPT, EPT)], idx_v)
    vbufs = (v0, v1)
    sems = (sm0, sm1)

    def _pass(m_hbm, lane0, agg_hbm, olane0, wide):
        for ro, rn in _ZCH:
            pltpu.sync_copy(zbuf.at[pl.ds(0, rn)],
                            acc.at[pl.ds(sid * RPT + ro, rn)])
        plsc.subcore_barrier()

        def load(i):
            off = sid * EPT + i * CS
            return pltpu.async_copy(
                m_hbm.at[pl.ds(off, CS), pl.ds(lane0, 16)], vbufs[i & 1],
                sems[i & 1])

        pend = load(0)
        for i in range(nch):
            nxt = load(i + 1) if i + 1 < nch else None
            pend.wait()
            pltpu.sync_copy(vbufs[i & 1], acc.at[idx_v.at[pl.ds(i * CS, CS)]],
                            add=True)
            pend = nxt

        plsc.subcore_barrier()
        for ro, rn in _ZCH:
            s = pl.ds(sid * RPT + ro, rn)
            if wide:
                pltpu.sync_copy(acc.at[s], agg_hbm.at[s, pl.ds(olane0, 16)])
            else:
                pltpu.sync_copy(acc.at[s], agg_hbm.at[s])
        plsc.subcore_barrier()

    @pl.when(cid == 0)
    def _core0():
        for g in range(4):
            _pass(ma_hbm, 16 * g, agga_hbm, 16 * g, True)

    @pl.when(cid == 1)
    def _core1():
        for g in range(4, 8):
            _pass(ma_hbm, 16 * g, agga_hbm, 16 * g, True)
        _pass(mb_hbm, 0, aggb_hbm, 0, None)


@functools.cache
def _scatter():
    return pl.kernel(
        _scatter_body,
        out_type=[jax.ShapeDtypeStruct((N_PAD, C), _f32),
                  jax.ShapeDtypeStruct((N_PAD, 16), _f32)],
        mesh=plsc.VectorSubcoreMesh(core_axis_name="c", subcore_axis_name="s"),
        compiler_params=_SC_PARAMS,
        scratch_types=[
            pltpu.VMEM((EPT,), jnp.int32),
            pltpu.VMEM((CS, 16), _f32),
            pltpu.VMEM((CS, 16), _f32),
            pltpu.VMEM((1024, 16), _f32),
            pltpu.VMEM_SHARED((N_PAD, 16), _f32),
            pltpu.SemaphoreType.DMA,
            pltpu.SemaphoreType.DMA,
        ],
    )


def _edge_body(src_ref, dst_ref, ep_ref, a1_ref, b1m_ref, w1s_ref, w2a_ref,
               c_ref, ma_ref, mb_ref):
    src = src_ref[...]
    dst = dst_ref[...]
    srcf = src
    e0 = ep_ref[:, 3:4]
    e1 = ep_ref[:, 4:5]

    def row(i):
        return c_ref[i:i + 1, :]

    def scal(k):
        return c_ref[14:15, k:k + 1]

    dstf = dst

    z = (jnp.dot(src, a1_ref[...], preferred_element_type=_f32)
         + jnp.dot(dst, b1m_ref[...], preferred_element_type=_f32)
         + e0 * row(4) + e1 * row(5) + row(0))
    h = jnp.maximum(z, 0.0)
    d_pre = h * row(1) + srcf * row(2) + dstf * row(3)
    d = 0.5 * (jnp.sum(d_pre, axis=1, keepdims=True)
               + e0 * scal(1) + e1 * scal(2) + scal(0))
    s = d * lax.rsqrt(d * d + 1e-5)
    e0n = e0 + scal(3) * s + scal(5)
    e1n = e1 - scal(4) * s + scal(6)

    zz = (jnp.dot(src, w1s_ref[...], preferred_element_type=_f32)
          + e0n * row(6) + e1n * row(7) + row(8))
    hhf = jnp.maximum(zz, 0.0)
    hh = hhf
    y = srcf + jnp.dot(hh, w2a_ref[...], preferred_element_type=_f32) + row(9)
    y0 = e0n + jnp.sum(hhf * row(12), axis=1, keepdims=True) + scal(7)
    y1 = e1n + jnp.sum(hhf * row(13), axis=1, keepdims=True) + scal(8)
    tot = float(C + EC)
    sy = jnp.sum(y + y * y, axis=1, keepdims=True)
    sy2 = jnp.sum((y - 1.0) * y, axis=1, keepdims=True)
    mean = (0.5 * (sy - sy2) + y0 + y1) / tot
    var = (0.5 * (sy + sy2) + y0 * y0 + y1 * y1) / tot - mean * mean
    inv = lax.rsqrt(var + 1e-5)
    ma_ref[...] = (y - mean) * inv * row(10) + row(11)
    o0 = (y0 - mean) * inv * scal(9) + scal(11)
    o1 = (y1 - mean) * inv * scal(10) + scal(12)
    mb_ref[...] = jnp.concatenate(
        [o0, o1, jnp.ones_like(o0), e0n, e1n,
         jnp.zeros((EB, 27), _f32)], axis=1)


_edge_call = pl.pallas_call(
    _edge_body,
    grid=(E_PAD // EB,),
    in_specs=[
        pl.BlockSpec((EB, C), lambda i: (i, 0)),
        pl.BlockSpec((EB, C), lambda i: (i, 0)),
        pl.BlockSpec((EB, 32), lambda i: (i, 0)),
        pl.BlockSpec((C, C), lambda i: (0, 0)),
        pl.BlockSpec((C, C), lambda i: (0, 0)),
        pl.BlockSpec((C, C), lambda i: (0, 0)),
        pl.BlockSpec((C, C), lambda i: (0, 0)),
        pl.BlockSpec((16, C), lambda i: (0, 0)),
    ],
    out_specs=[pl.BlockSpec((EB, C), lambda i: (i, 0)),
               pl.BlockSpec((EB, 32), lambda i: (i, 0))],
    out_shape=[jax.ShapeDtypeStruct((E_PAD, C), _f32),
               jax.ShapeDtypeStruct((E_PAD, 32), _f32)],
)


def _node_body(n_ref, aa_ref, ab_ref, w1a_ref, w1b_ref, w2_ref, rwa_ref,
               rwb_ref, c_ref, out_ref, outb_ref):
    nodes = n_ref[...]

    def row(i):
        return c_ref[i:i + 1, :]

    inv_d = 1.0 / jnp.maximum(ab_ref[:, 2:3], 1.0)
    aggc = aa_ref[...] * inv_d
    ae0 = ab_ref[:, 0:1] * inv_d
    ae1 = ab_ref[:, 1:2] * inv_d

    z = (jnp.dot(nodes, w1a_ref[...], preferred_element_type=_f32)
         + jnp.dot(aggc, w1b_ref[...], preferred_element_type=_f32)
         + ae0 * row(5) + ae1 * row(6) + row(0))
    hh = jnp.dot(jnp.maximum(z, 0.0), w2_ref[...],
                 preferred_element_type=_f32) + row(1)
    r = (jnp.dot(nodes, rwa_ref[...], preferred_element_type=_f32)
         + jnp.dot(aggc, rwb_ref[...], preferred_element_type=_f32)
         + ae0 * row(7) + ae1 * row(8) + row(2))
    y = r + hh
    mean = jnp.mean(y, axis=1, keepdims=True)
    var = jnp.mean((y - mean) ** 2, axis=1, keepdims=True)
    out = (y - mean) * lax.rsqrt(var + 1e-5) * row(3) + row(4) + nodes
    out_ref[...] = out
    outb_ref[...] = out.astype(_bf16)


_node_call = pl.pallas_call(
    _node_body,
    grid=(N_PAD // NB,),
    in_specs=[
        pl.BlockSpec((NB, C), lambda i: (i, 0)),
        pl.BlockSpec((NB, C), lambda i: (i, 0)),
        pl.BlockSpec((NB, 16), lambda i: (i, 0)),
        pl.BlockSpec((C, C), lambda i: (0, 0)),
        pl.BlockSpec((C, C), lambda i: (0, 0)),
        pl.BlockSpec((C, C), lambda i: (0, 0)),
        pl.BlockSpec((C, C), lambda i: (0, 0)),
        pl.BlockSpec((C, C), lambda i: (0, 0)),
        pl.BlockSpec((16, C), lambda i: (0, 0)),
    ],
    out_specs=[pl.BlockSpec((NB, C), lambda i: (i, 0))] * 2,
    out_shape=[jax.ShapeDtypeStruct((N_PAD, C), _f32),
               jax.ShapeDtypeStruct((N_PAD, C), _bf16)],
)


def kernel(x, edge_index, edge_attr, em_res_w, em_res_b, em_w1, em_b1, em_w2,
           em_b2, em_ln_g, em_ln_b, n1_w1, n1_b1, n1_w2, n1_b2, n1_ln_g,
           n1_ln_b, n2_res_w, n2_res_b, n2_w1, n2_b1, n2_w2, n2_b2, n2_ln_g,
           n2_ln_b):
    pad = N_NODES - N_INPUT
    nodes = jnp.zeros((N_PAD, C), _f32).at[pad:N_NODES].set(x)
    nodes_b = nodes
    row_pad = jnp.concatenate(
        [edge_index[0].astype(jnp.int32),
         jnp.zeros((E_PAD - N_EDGES,), jnp.int32)])
    col_pad = jnp.concatenate(
        [edge_index[1].astype(jnp.int32),
         jnp.full((E_PAD - N_EDGES,), DUMP, jnp.int32)])
    ep = jnp.zeros((E_PAD, 32), _f32).at[:N_EDGES, 3:5].set(edge_attr)

    for i in range(L):
        wd = em_w2[i][:, 0] - em_w2[i][:, 1]
        rd = em_res_w[i][:, 0] - em_res_w[i][:, 1]
        cd = (em_b2[i][0] - em_b2[i][1]) + (em_res_b[i][0] - em_res_b[i][1])
        ce = jnp.zeros((16, C), _f32)
        ce = ce.at[0].set(em_b1[i])
        ce = ce.at[1].set(wd)
        ce = ce.at[2].set(rd[:C])
        ce = ce.at[3].set(rd[C:2 * C])
        ce = ce.at[4].set(em_w1[i][2 * C])
        ce = ce.at[5].set(em_w1[i][2 * C + 1])
        ce = ce.at[6].set(n1_w1[i][C])
        ce = ce.at[7].set(n1_w1[i][C + 1])
        ce = ce.at[8].set(n1_b1[i])
        ce = ce.at[9].set(n1_b2[i][:C])
        ce = ce.at[10].set(n1_ln_g[i][:C])
        ce = ce.at[11].set(n1_ln_b[i][:C])
        ce = ce.at[12].set(n1_w2[i][:, C])
        ce = ce.at[13].set(n1_w2[i][:, C + 1])
        sc = jnp.stack([
            cd, rd[2 * C], rd[2 * C + 1], em_ln_g[i][0], em_ln_g[i][1],
            em_ln_b[i][0], em_ln_b[i][1], n1_b2[i][C], n1_b2[i][C + 1],
            n1_ln_g[i][C], n1_ln_g[i][C + 1], n1_ln_b[i][C],
            n1_ln_b[i][C + 1],
        ])
        ce = ce.at[14, :13].set(sc)

        cn = jnp.zeros((16, C), _f32)
        cn = cn.at[0].set(n2_b1[i])
        cn = cn.at[1].set(n2_b2[i])
        cn = cn.at[2].set(n2_res_b[i])
        cn = cn.at[3].set(n2_ln_g[i])
        cn = cn.at[4].set(n2_ln_b[i])
        cn = cn.at[5].set(n2_w1[i][2 * C])
        cn = cn.at[6].set(n2_w1[i][2 * C + 1])
        cn = cn.at[7].set(n2_res_w[i][2 * C])
        cn = cn.at[8].set(n2_res_w[i][2 * C + 1])

        src, dst = _gather()(nodes, row_pad, col_pad)
        ma, mb = _edge_call(src, dst, ep, em_w1[i][:C], em_w1[i][C:2 * C],
                            n1_w1[i][:C], n1_w2[i][:, :C], ce)
        agga, aggb = _scatter()(ma, mb, col_pad)
        nodes, nodes_b = _node_call(nodes, agga, aggb, n2_w1[i][:C],
                                    n2_w1[i][C:2 * C], n2_w2[i],
                                    n2_res_w[i][:C], n2_res_w[i][C:2 * C], cn)
        ep = mb

    return nodes[pad:N_NODES]
